# baseline (speedup 1.0000x reference)
import functools

import jax
import jax.numpy as jnp
from jax import lax
from jax.experimental import pallas as pl
from jax.experimental.pallas import tpu as pltpu
from jax.experimental.pallas import tpu_sc as plsc

D = 1024
C = 16
NC = 2
NS = 16
NW = NC * NS
L = 16
UNROLL = 16
NXB = 5
NPB = 2
CPR = D // L

B_SC = 2
BS = 512


def _sc_body(x_hbm, pos_hbm, out_hbm, *scratch):
    xv = scratch[0:NXB]
    pv = scratch[NXB:NXB + NPB]
    xs = scratch[NXB + NPB:NXB + NPB + NXB]
    os_ = scratch[NXB + NPB + NXB:NXB + NPB + 2 * NXB]
    ps = scratch[NXB + NPB + 2 * NXB:]

    nbatch = out_hbm.shape[0]
    boff = x_hbm.shape[0] - nbatch
    s = x_hbm.shape[1]
    seq_per_w = s // NW
    nchunks = seq_per_w // C
    nitems = nchunks * nbatch

    wid = lax.axis_index("s") * NC + lax.axis_index("c")
    w0 = wid * seq_per_w

    def start_xload(k):
        it, b = divmod(k, nbatch)
        j = k % NXB
        return pltpu.async_copy(
            x_hbm.at[boff + b, pl.ds(w0 + it * C, C)], xv[j], xs[j])

    def start_posload(it):
        j = it % NPB
        return pltpu.async_copy(
            pos_hbm.at[pl.ds(w0 + it * C, C)], pv[j], ps[j])

    pos_loads = {0: start_posload(0)}
    x_loads = {k: start_xload(k) for k in range(min(NXB - 1, nitems))}
    last_store = [None] * NXB

    for k in range(nitems):
        it, b = divmod(k, nbatch)
        j = k % NXB
        if b == 0:
            pos_loads.pop(it).wait()
            if it + 1 < nchunks:
                pos_loads[it + 1] = start_posload(it + 1)
        x_loads.pop(k).wait()

        xbuf = xv[j]
        pbuf = pv[it % NPB]

        @plsc.parallel_loop(0, C * CPR, step=1, unroll=UNROLL)
        def _(n):
            r = lax.shift_right_logical(n, 6)
            c = pl.multiple_of(lax.shift_left(lax.bitwise_and(n, CPR - 1), 4), L)
            plsc.addupdate(xbuf.at[r, pl.ds(c, L)], pbuf[r, pl.ds(c, L)])

        last_store[j] = pltpu.async_copy(
            xbuf, out_hbm.at[b, pl.ds(w0 + it * C, C)], os_[j])

        n = k + NXB - 1
        if n < nitems:
            jn = n % NXB
            if last_store[jn] is not None:
                last_store[jn].wait()
                last_store[jn] = None
            x_loads[n] = start_xload(n)

    for st in last_store:
        if st is not None:
            st.wait()


def _sc_run(x, pos_table, b_sc):
    b, s, d = x.shape
    mesh = plsc.VectorSubcoreMesh(core_axis_name="c", subcore_axis_name="s")
    run = functools.partial(
        pl.kernel,
        mesh=mesh,
        out_type=jax.ShapeDtypeStruct((b_sc, s, d), jnp.float32),
        scratch_types=(
            [pltpu.VMEM((C, D), jnp.float32) for _ in range(NXB)]
            + [pltpu.VMEM((C, D), jnp.float32) for _ in range(NPB)]
            + [pltpu.SemaphoreType.DMA for _ in range(2 * NXB + NPB)]
        ),
    )(_sc_body)
    return run(x, pos_table)


def _tc_body(x_ref, pos_ref, out_ref):
    out_ref[...] = x_ref[...] + pos_ref[...][None]


def _tc_run(x, pos_table, b_tc):
    b, s, d = x.shape
    return pl.pallas_call(
        _tc_body,
        grid=(s // BS, b_tc),
        in_specs=[
            pl.BlockSpec((1, BS, d), lambda i, j: (j, i, 0)),
            pl.BlockSpec((BS, d), lambda i, j: (i, 0)),
        ],
        out_specs=pl.BlockSpec((1, BS, d), lambda i, j: (j, i, 0)),
        out_shape=jax.ShapeDtypeStruct((b_tc, s, d), x.dtype),
    )(x, pos_table)


def kernel(x, pos_table):
    b = x.shape[0]
    sc_out = _sc_run(x, pos_table, B_SC)
    tc_out = _tc_run(x, pos_table, b - B_SC)
    return jnp.concatenate([tc_out, sc_out], axis=0)


# speedup vs baseline: 1.5954x; 1.5954x over previous
import functools

import jax
import jax.numpy as jnp
from jax import lax
from jax.experimental import pallas as pl
from jax.experimental.pallas import tpu as pltpu
from jax.experimental.pallas import tpu_sc as plsc

D = 1024
C = 16
NC = 2
NS = 16
NW = NC * NS
L = 16
UNROLL = 8
NXB = 5
NPB = 2
CPR = D // L
NH = 2
CH = C // NH


def _sc_body(x_hbm, pos_hbm, out_hbm, *scratch):
    xv = scratch[0:NXB]
    pv = scratch[NXB:NXB + NPB]
    xs = scratch[NXB + NPB:NXB + NPB + NXB]
    os_ = scratch[NXB + NPB + NXB:NXB + NPB + 2 * NXB]
    ps = scratch[NXB + NPB + 2 * NXB:]

    nbatch, s, _ = x_hbm.shape
    seq_per_w = s // NW
    nchunks = seq_per_w // C
    nitems = nchunks * nbatch

    wid = lax.axis_index("s") * NC + lax.axis_index("c")
    w0 = wid * seq_per_w

    def start_xload(k):
        it, b = divmod(k, nbatch)
        j = k % NXB
        return pltpu.async_copy(
            x_hbm.at[b, pl.ds(w0 + it * C, C)], xv[j], xs[j])

    def start_posload(it):
        j = it % NPB
        return pltpu.async_copy(
            pos_hbm.at[pl.ds(w0 + it * C, C)], pv[j], ps[j])

    pos_loads = {0: start_posload(0)}
    x_loads = {k: start_xload(k) for k in range(min(NXB - 1, nitems))}
    last_stores = [[] for _ in range(NXB)]

    for k in range(nitems):
        it, b = divmod(k, nbatch)
        j = k % NXB
        if b == 0:
            pos_loads.pop(it).wait()
            if it + 1 < nchunks:
                pos_loads[it + 1] = start_posload(it + 1)
        x_loads.pop(k).wait()

        xbuf = xv[j]
        pbuf = pv[it % NPB]

        stores = []
        for h in range(NH):
            @plsc.parallel_loop(0, CH * CPR, step=1, unroll=UNROLL)
            def _(n, _h=h):
                r = _h * CH + lax.shift_right_logical(n, 6)
                c = pl.multiple_of(
                    lax.shift_left(lax.bitwise_and(n, CPR - 1), 4), L)
                plsc.addupdate(xbuf.at[r, pl.ds(c, L)], pbuf[r, pl.ds(c, L)])

            stores.append(pltpu.async_copy(
                xbuf.at[pl.ds(h * CH, CH)],
                out_hbm.at[b, pl.ds(w0 + it * C + h * CH, CH)],
                os_[j]))
        last_stores[j] = stores

        n = k + NXB - 1
        if n < nitems:
            jn = n % NXB
            for st in last_stores[jn]:
                st.wait()
            last_stores[jn] = []
            x_loads[n] = start_xload(n)

    for stores in last_stores:
        for st in stores:
            st.wait()


def kernel(x, pos_table):
    b, s, d = x.shape

    mesh = plsc.VectorSubcoreMesh(core_axis_name="c", subcore_axis_name="s")
    run = functools.partial(
        pl.kernel,
        mesh=mesh,
        out_type=jax.ShapeDtypeStruct((b, s, d), jnp.float32),
        scratch_types=(
            [pltpu.VMEM((C, D), jnp.float32) for _ in range(NXB)]
            + [pltpu.VMEM((C, D), jnp.float32) for _ in range(NPB)]
            + [pltpu.SemaphoreType.DMA for _ in range(2 * NXB + NPB)]
        ),
    )(_sc_body)
    return run(x, pos_table)


# speedup vs baseline: 1.5992x; 1.0024x over previous
import functools

import jax
import jax.numpy as jnp
from jax import lax
from jax.experimental import pallas as pl
from jax.experimental.pallas import tpu as pltpu
from jax.experimental.pallas import tpu_sc as plsc

D = 1024
C = 16
NC = 2
NS = 16
NW = NC * NS
L = 16
UNROLL = 8
NXB = 5
NPB = 2
CPR = D // L
NH = 2
CH = C // NH


def _sc_body(x_hbm, pos_hbm, out_hbm, *scratch):
    xv = scratch[0:NXB]
    pv = scratch[NXB:NXB + NPB]
    xs = scratch[NXB + NPB:NXB + NPB + NXB]
    os_ = scratch[NXB + NPB + NXB:NXB + NPB + 2 * NXB]
    ps = scratch[NXB + NPB + 2 * NXB:]

    nbatch, s, _ = x_hbm.shape
    seq_per_w = s // NW
    nchunks = seq_per_w // C
    nitems = nchunks * nbatch

    wid = lax.axis_index("s") * NC + lax.axis_index("c")
    w0 = wid * seq_per_w

    def start_xload(k):
        it, b = divmod(k, nbatch)
        j = k % NXB
        return pltpu.async_copy(
            x_hbm.at[b, pl.ds(w0 + it * C, C)], xv[j], xs[j])

    def start_posload(it):
        j = it % NPB
        return pltpu.async_copy(
            pos_hbm.at[pl.ds(w0 + it * C, C)], pv[j], ps[j])

    pos_loads = {0: start_posload(0)}
    x_loads = {k: start_xload(k) for k in range(min(NXB - 1, nitems))}
    last_stores = [[] for _ in range(NXB)]

    for k in range(nitems):
        it, b = divmod(k, nbatch)
        j = k % NXB
        if b == 0:
            pos_loads.pop(it).wait()
            if it + 1 < nchunks:
                pos_loads[it + 1] = start_posload(it + 1)
        x_loads.pop(k).wait()

        xbuf = xv[j]
        pbuf = pv[it % NPB]

        stores = []
        for h in range(NH):
            @plsc.parallel_loop(0, CH * CPR, step=1, unroll=UNROLL)
            def _(n, _h=h):
                r = _h * CH + lax.shift_right_logical(n, 6)
                c = pl.multiple_of(
                    lax.shift_left(lax.bitwise_and(n, CPR - 1), 4), L)
                plsc.addupdate(xbuf.at[r, pl.ds(c, L)], pbuf[r, pl.ds(c, L)])

            stores.append(pltpu.async_copy(
                xbuf.at[pl.ds(h * CH, CH)],
                out_hbm.at[b, pl.ds(w0 + it * C + h * CH, CH)],
                os_[j]))
            if h == 0:
                n = k + NXB - 1
                if n < nitems:
                    jn = n % NXB
                    for st in last_stores[jn]:
                        st.wait()
                    last_stores[jn] = []
                    x_loads[n] = start_xload(n)
        last_stores[j] = stores

    for stores in last_stores:
        for st in stores:
            st.wait()


def kernel(x, pos_table):
    b, s, d = x.shape

    mesh = plsc.VectorSubcoreMesh(core_axis_name="c", subcore_axis_name="s")
    run = functools.partial(
        pl.kernel,
        mesh=mesh,
        out_type=jax.ShapeDtypeStruct((b, s, d), jnp.float32),
        scratch_types=(
            [pltpu.VMEM((C, D), jnp.float32) for _ in range(NXB)]
            + [pltpu.VMEM((C, D), jnp.float32) for _ in range(NPB)]
            + [pltpu.SemaphoreType.DMA for _ in range(2 * NXB + NPB)]
        ),
    )(_sc_body)
    return run(x, pos_table)
